# baseline (speedup 1.0000x reference)
import jax
import jax.numpy as jnp
from jax.experimental import pallas as pl

_P_TILE = 256


def _pe_add_kernel(x_ref, pe_ref, o_ref):
    o_ref[0] = x_ref[0].T + pe_ref[...]


def kernel(image_feature, pe_table):
    B, C, H, W = image_feature.shape
    P = H * W
    x = image_feature.reshape(B, C, P)
    grid = (B, P // _P_TILE)
    return pl.pallas_call(
        _pe_add_kernel,
        grid=grid,
        in_specs=[
            pl.BlockSpec((1, C, _P_TILE), lambda b, j: (b, 0, j)),
            pl.BlockSpec((_P_TILE, C), lambda b, j: (j, 0)),
        ],
        out_specs=pl.BlockSpec((1, _P_TILE, C), lambda b, j: (b, j, 0)),
        out_shape=jax.ShapeDtypeStruct((B, P, C), image_feature.dtype),
    )(x, pe_table)


# speedup vs baseline: 1.0855x; 1.0855x over previous
import jax
import jax.numpy as jnp
from jax.experimental import pallas as pl

_P_TILE = 256


def _pe_add_kernel(x_ref, pe_ref, o_ref):
    o_ref[0] = x_ref[0].T + pe_ref[...]


def kernel(image_feature, pe_table):
    B, C, H, W = image_feature.shape
    P = H * W
    x = image_feature.reshape(B, C, P)
    grid = (P // _P_TILE, B)
    return pl.pallas_call(
        _pe_add_kernel,
        grid=grid,
        in_specs=[
            pl.BlockSpec((1, C, _P_TILE), lambda j, b: (b, 0, j)),
            pl.BlockSpec((_P_TILE, C), lambda j, b: (j, 0)),
        ],
        out_specs=pl.BlockSpec((1, _P_TILE, C), lambda j, b: (b, j, 0)),
        out_shape=jax.ShapeDtypeStruct((B, P, C), image_feature.dtype),
    )(x, pe_table)


# speedup vs baseline: 1.3088x; 1.2057x over previous
import jax
import jax.numpy as jnp
from jax.experimental import pallas as pl

_P_TILE = 512


def _pe_add_kernel(x_ref, pe_ref, o_ref):
    o_ref[0] = x_ref[0].T + pe_ref[...]


def kernel(image_feature, pe_table):
    B, C, H, W = image_feature.shape
    P = H * W
    x = image_feature.reshape(B, C, P)
    grid = (P // _P_TILE, B)
    return pl.pallas_call(
        _pe_add_kernel,
        grid=grid,
        in_specs=[
            pl.BlockSpec((1, C, _P_TILE), lambda j, b: (b, 0, j)),
            pl.BlockSpec((_P_TILE, C), lambda j, b: (j, 0)),
        ],
        out_specs=pl.BlockSpec((1, _P_TILE, C), lambda j, b: (b, j, 0)),
        out_shape=jax.ShapeDtypeStruct((B, P, C), image_feature.dtype),
    )(x, pe_table)


# speedup vs baseline: 1.4541x; 1.1110x over previous
import jax
import jax.numpy as jnp
from jax.experimental import pallas as pl

_P_TILE = 1024


def _pe_add_kernel(x_ref, pe_ref, o_ref):
    o_ref[0] = x_ref[0].T + pe_ref[...]


def kernel(image_feature, pe_table):
    B, C, H, W = image_feature.shape
    P = H * W
    x = image_feature.reshape(B, C, P)
    grid = (P // _P_TILE, B)
    return pl.pallas_call(
        _pe_add_kernel,
        grid=grid,
        in_specs=[
            pl.BlockSpec((1, C, _P_TILE), lambda j, b: (b, 0, j)),
            pl.BlockSpec((_P_TILE, C), lambda j, b: (j, 0)),
        ],
        out_specs=pl.BlockSpec((1, _P_TILE, C), lambda j, b: (b, j, 0)),
        out_shape=jax.ShapeDtypeStruct((B, P, C), image_feature.dtype),
    )(x, pe_table)
